# baseline (speedup 1.0000x reference)
import functools

import jax
import jax.numpy as jnp
from jax.experimental import pallas as pl
from jax.experimental.pallas import tpu as pltpu

M_BLK = 400


def _h_kernel(x_ref, w_ref, h_ref):
    h_ref[...] = jnp.dot(x_ref[...], w_ref[...], preferred_element_type=jnp.float32)


def _adj_kernel(adj_ref, h_ref, b_ref, out_ref):
    out_ref[...] = (
        jnp.dot(adj_ref[...], h_ref[...], preferred_element_type=jnp.float32)
        + b_ref[...]
    )


@jax.jit
def kernel(x, adj, w, b):
    n, in_dim = x.shape
    out_dim = w.shape[1]
    b2 = b.reshape(1, out_dim)
    h = pl.pallas_call(
        _h_kernel,
        out_shape=jax.ShapeDtypeStruct((n, out_dim), jnp.float32),
    )(x, w)
    out = pl.pallas_call(
        _adj_kernel,
        grid=(pl.cdiv(n, M_BLK),),
        in_specs=[
            pl.BlockSpec((M_BLK, n), lambda i: (i, 0)),
            pl.BlockSpec((n, out_dim), lambda i: (0, 0)),
            pl.BlockSpec((1, out_dim), lambda i: (0, 0)),
        ],
        out_specs=pl.BlockSpec((M_BLK, out_dim), lambda i: (i, 0)),
        out_shape=jax.ShapeDtypeStruct((n, out_dim), jnp.float32),
        compiler_params=pltpu.CompilerParams(
            dimension_semantics=("parallel",),
        ),
    )(adj, h, b2)
    return out


# speedup vs baseline: 1.0054x; 1.0054x over previous
import functools

import jax
import jax.numpy as jnp
from jax.experimental import pallas as pl
from jax.experimental.pallas import tpu as pltpu

M_BLK = 400
NBUF = 3


def _gcn_kernel(adj_hbm, x_ref, w_ref, b_ref, out_ref, h_ref, bufs, sems):
    i = pl.program_id(0)
    nblk = pl.num_programs(0)

    def start_copy(blk, slot):
        pltpu.make_async_copy(
            adj_hbm.at[pl.ds(blk * M_BLK, M_BLK), :],
            bufs.at[slot],
            sems.at[slot],
        ).start()

    @pl.when(i == 0)
    def _():
        for k in range(NBUF):
            start_copy(k, k)
        h_ref[...] = jnp.dot(
            x_ref[...], w_ref[...], preferred_element_type=jnp.float32
        )

    slot = jax.lax.rem(i, NBUF)
    pltpu.make_async_copy(
        adj_hbm.at[pl.ds(i * M_BLK, M_BLK), :],
        bufs.at[slot],
        sems.at[slot],
    ).wait()
    out_ref[...] = (
        jnp.dot(bufs[slot], h_ref[...], preferred_element_type=jnp.float32)
        + b_ref[...]
    )
    nxt = i + NBUF

    @pl.when(nxt < nblk)
    def _():
        start_copy(nxt, slot)


@jax.jit
def kernel(x, adj, w, b):
    n, in_dim = x.shape
    out_dim = w.shape[1]
    b2 = b.reshape(1, out_dim)
    out = pl.pallas_call(
        _gcn_kernel,
        grid=(n // M_BLK,),
        in_specs=[
            pl.BlockSpec(memory_space=pltpu.MemorySpace.HBM),
            pl.BlockSpec((n, in_dim), lambda i: (0, 0)),
            pl.BlockSpec((in_dim, out_dim), lambda i: (0, 0)),
            pl.BlockSpec((1, out_dim), lambda i: (0, 0)),
        ],
        out_specs=pl.BlockSpec((M_BLK, out_dim), lambda i: (i, 0)),
        out_shape=jax.ShapeDtypeStruct((n, out_dim), jnp.float32),
        scratch_shapes=[
            pltpu.VMEM((n, out_dim), jnp.float32),
            pltpu.VMEM((NBUF, M_BLK, n), jnp.float32),
            pltpu.SemaphoreType.DMA((NBUF,)),
        ],
    )(adj, x, w, b2)
    return out
